# Initial kernel scaffold; baseline (speedup 1.0000x reference)
import jax
import jax.numpy as jnp
from jax.experimental import pallas as pl


def kernel(targets, W, b):
    raise NotImplementedError("write your pallas kernel here")


# speedup vs baseline: 4.5772x; 4.5772x over previous
import jax
import jax.numpy as jnp
from jax import lax
from jax.experimental import pallas as pl
from jax.experimental.pallas import tpu as pltpu

_KB = 64


def _encoder_kernel(wt_ref, tt_ref, bt_ref, out_ref, cur_ref, bl_ref, bv_ref):
    s = pl.program_id(0)
    kb = pl.program_id(1)
    nkb = pl.num_programs(1)
    d = tt_ref.shape[0]
    n = tt_ref.shape[1]

    @pl.when(jnp.logical_and(s == 0, kb == 0))
    def _init_current():
        cur_ref[...] = jnp.zeros((d, n), jnp.float32)

    @pl.when(kb == 0)
    def _init_best():
        bl_ref[...] = jnp.full((1, n), jnp.inf, jnp.float32)

    outs = jnp.dot(wt_ref[0], cur_ref[...], preferred_element_type=jnp.float32)
    outs = outs + bt_ref[0]
    outs3 = outs.reshape(_KB, d, n)

    diff = outs3 - tt_ref[...][None, :, :]
    losses = jnp.sum(diff * diff, axis=1)

    bmin = jnp.min(losses, axis=0)
    kiota = lax.broadcasted_iota(jnp.int32, (_KB, n), 0)
    bidx = jnp.min(jnp.where(losses <= bmin[None, :], kiota, _KB), axis=0)
    onehot = (kiota == bidx[None, :]).astype(jnp.float32)
    bvec = jnp.sum(outs3 * onehot[:, None, :], axis=0)

    prev = bl_ref[...]
    better = bmin[None, :] < prev
    bl_ref[...] = jnp.where(better, bmin[None, :], prev)
    bv_ref[...] = jnp.where(better, bvec, bv_ref[...])

    @pl.when(kb == nkb - 1)
    def _finish_stage():
        cur_ref[...] = bv_ref[...]
        out_ref[0] = bv_ref[...]


def kernel(targets, W, b):
    num_stages, psize, kd = W.shape
    batch = targets.shape[0]
    nkb = (kd // psize) // _KB
    kbs = _KB * psize

    wt = W.transpose(0, 2, 1)
    tt = targets.T
    bt = b.reshape(num_stages, kd, 1)

    out_t = pl.pallas_call(
        _encoder_kernel,
        grid=(num_stages, nkb),
        in_specs=[
            pl.BlockSpec((1, kbs, psize), lambda s, kb: (s, kb, 0)),
            pl.BlockSpec((psize, batch), lambda s, kb: (0, 0)),
            pl.BlockSpec((1, kbs, 1), lambda s, kb: (s, kb, 0)),
        ],
        out_specs=pl.BlockSpec((1, psize, batch), lambda s, kb: (s, 0, 0)),
        out_shape=jax.ShapeDtypeStruct((num_stages, psize, batch), jnp.float32),
        scratch_shapes=[
            pltpu.VMEM((psize, batch), jnp.float32),
            pltpu.VMEM((1, batch), jnp.float32),
            pltpu.VMEM((psize, batch), jnp.float32),
        ],
        compiler_params=pltpu.CompilerParams(
            dimension_semantics=("arbitrary", "arbitrary"),
        ),
    )(wt, tt, bt)

    return out_t.transpose(2, 0, 1)
